# baseline (speedup 1.0000x reference)
import jax
import jax.numpy as jnp
from jax.experimental import pallas as pl
from jax.experimental.pallas import tpu as pltpu

NUM_TYPES = 13
LPS = 8
NIB = 24
NOB = 16


def _in_copy(px_hbm, px2, isem, j):
    return pltpu.make_async_copy(
        px_hbm.at[:, j, :], px2.at[j % NIB], isem.at[j % NIB]
    )


def _out_copy(scratch, out_hbm, osem, j):
    return pltpu.make_async_copy(
        scratch.at[j % NOB], out_hbm.at[:, j, :], osem.at[j % NOB]
    )


def _seq_embed_kernel(tt_ref, rhs_ref, px_hbm, out_hbm, px2, scratch, isem, osem):
    s = pl.program_id(0)
    ns = pl.num_programs(0)
    l0 = s * LPS
    n = ns * LPS
    B = px_hbm.shape[2]

    @pl.when(s == 0)
    def _prologue():
        for j in range(2 * LPS):
            _in_copy(px_hbm, px2, isem, j).start()

    for i in range(LPS):
        jpf = l0 + i + 2 * LPS

        @pl.when(jpf < n)
        def _prefetch(jpf=jpf):
            _in_copy(px_hbm, px2, isem, jpf).start()

    for i in range(LPS):
        l = l0 + i
        lo = l % 8

        @pl.when(l >= NOB)
        def _wait_out(l=l):
            _out_copy(scratch, out_hbm, osem, l - NOB).wait()

        _in_copy(px_hbm, px2, isem, l).wait()
        x = jnp.maximum(px2[l % NIB], 0.0)
        t = tt_ref[pl.ds(lo, 1), :]
        safe_t = jnp.where(t < 0, NUM_TYPES - 1, jnp.minimum(t, NUM_TYPES - 1))
        iota_t = jax.lax.broadcasted_iota(jnp.int32, (NUM_TYPES, B), 0)
        onehot = (iota_t == safe_t).astype(jnp.float32)
        ones = jnp.ones((1, B), jnp.float32)
        lhs = jnp.concatenate([x, onehot, ones], axis=0)
        y = jax.lax.dot_general(
            lhs, rhs_ref[i],
            dimension_numbers=(((0,), (0,)), ((), ())),
            preferred_element_type=jnp.float32,
            precision=jax.lax.Precision.DEFAULT,
        )
        scratch[l % NOB] = y
        _out_copy(scratch, out_hbm, osem, l).start()

    @pl.when(s == ns - 1)
    def _drain():
        for k in range(NOB):
            _out_copy(scratch, out_hbm, osem, n - 1 - k).wait()


def kernel(entity_type, entity_params, entity_embed_w, param_fc_w, param_fc_b, pos_embed_w):
    B, L = entity_type.shape
    P = entity_params.shape[-1]
    D = param_fc_w.shape[-1]
    T = entity_embed_w.shape[0]
    params_t = jnp.transpose(entity_params, (2, 1, 0))
    type_t = jnp.transpose(entity_type, (1, 0))
    rhs = jnp.concatenate(
        [
            jnp.broadcast_to(param_fc_w[None], (L, P, D)),
            jnp.broadcast_to(entity_embed_w[None], (L, T, D)),
            (pos_embed_w[:L] + param_fc_b[None, :])[:, None, :],
        ],
        axis=1,
    )
    return pl.pallas_call(
        _seq_embed_kernel,
        grid=(L // LPS,),
        in_specs=[
            pl.BlockSpec((8, B), lambda s: (s * LPS // 8, 0)),
            pl.BlockSpec((LPS, P + T + 1, D), lambda s: (s, 0, 0)),
            pl.BlockSpec(memory_space=pl.ANY),
        ],
        out_specs=pl.BlockSpec(memory_space=pl.ANY),
        out_shape=jax.ShapeDtypeStruct((B, L, D), jnp.float32),
        scratch_shapes=[
            pltpu.VMEM((NIB, P, B), jnp.float32),
            pltpu.VMEM((NOB, B, D), jnp.float32),
            pltpu.SemaphoreType.DMA((NIB,)),
            pltpu.SemaphoreType.DMA((NOB,)),
        ],
    )(type_t, rhs, params_t)
